# Initial kernel scaffold; baseline (speedup 1.0000x reference)
import jax
import jax.numpy as jnp
from jax.experimental import pallas as pl


def kernel(token_ids, action_actors, action_streets, action_legal_masks, actor_emb_w, street_emb_w, action_type_emb_w, mlp_w, mlp_b, ln_gamma, ln_beta):
    raise NotImplementedError("write your pallas kernel here")


# speedup vs baseline: 3.5424x; 3.5424x over previous
import functools

import jax
import jax.numpy as jnp
from jax import lax
from jax.experimental import pallas as pl

_NUM_BET_BINS = 32
_D_MODEL = 128
_NUM_STREETS = 4
_ACTION_OFFSET = 10


def _fused_kernel(tok_ref, act_ref, st_ref, masks_ref, table_ref, mlp_w_ref,
                  mlp_b_ref, gamma_ref, beta_ref, out_ref):
    tok = tok_ref[...]
    act = act_ref[...]
    st = st_ref[...]

    r = tok.shape[0]
    valid = ((tok >= _ACTION_OFFSET)
             & (tok < _ACTION_OFFSET + _NUM_BET_BINS)).astype(jnp.float32)
    aid = jnp.clip(tok - _ACTION_OFFSET, 0, _NUM_BET_BINS - 1)
    act = jnp.clip(act, 0, 1)
    st = jnp.clip(st, 0, _NUM_STREETS - 1)

    i38 = lax.broadcasted_iota(jnp.int32, (r, 38), 1)
    oh = jnp.where(i38 < 2, (act == i38).astype(jnp.float32), 0.0)
    oh = jnp.where((i38 >= 2) & (i38 < 6),
                   (st == i38 - 2).astype(jnp.float32), oh)
    oh = jnp.where(i38 >= 6, (aid == i38 - 6).astype(jnp.float32), oh)

    emb = jnp.dot(oh, table_ref[...], preferred_element_type=jnp.float32)

    h = jnp.dot(masks_ref[...], mlp_w_ref[...],
                preferred_element_type=jnp.float32) + mlp_b_ref[...]
    m = jnp.mean(h, axis=1, keepdims=True)
    c = h - m
    v = jnp.mean(c * c, axis=1, keepdims=True)
    h = c * lax.rsqrt(v + 1e-5) * gamma_ref[...] + beta_ref[...]
    h = jnp.maximum(h, 0.0)

    out_ref[...] = valid * (emb + h)


@functools.partial(jax.jit, static_argnames=("block_rows",))
def _run(token_ids, action_actors, action_streets, action_legal_masks,
         table, mlp_w, mlp_b, ln_gamma, ln_beta, block_rows=2048):
    b, l = token_ids.shape
    n = b * l
    tok = token_ids.reshape(n, 1).astype(jnp.int32)
    act = action_actors.reshape(n, 1).astype(jnp.int32)
    st = action_streets.reshape(n, 1).astype(jnp.int32)
    masks = action_legal_masks.reshape(n, _NUM_BET_BINS)

    grid = n // block_rows
    row_spec = pl.BlockSpec((block_rows, 1), lambda i: (i, 0))
    full = lambda shape: pl.BlockSpec(shape, lambda i: (0, 0))

    out = pl.pallas_call(
        _fused_kernel,
        grid=(grid,),
        in_specs=[
            row_spec, row_spec, row_spec,
            pl.BlockSpec((block_rows, _NUM_BET_BINS), lambda i: (i, 0)),
            full(table.shape),
            full(mlp_w.shape),
            full((1, _D_MODEL)),
            full((1, _D_MODEL)),
            full((1, _D_MODEL)),
        ],
        out_specs=pl.BlockSpec((block_rows, _D_MODEL), lambda i: (i, 0)),
        out_shape=jax.ShapeDtypeStruct((n, _D_MODEL), jnp.float32),
    )(tok, act, st, masks, table, mlp_w,
      mlp_b.reshape(1, _D_MODEL), ln_gamma.reshape(1, _D_MODEL),
      ln_beta.reshape(1, _D_MODEL))
    return out.reshape(b, l, _D_MODEL)


def kernel(token_ids, action_actors, action_streets, action_legal_masks,
           actor_emb_w, street_emb_w, action_type_emb_w, mlp_w, mlp_b,
           ln_gamma, ln_beta):
    table = jnp.concatenate([actor_emb_w, street_emb_w, action_type_emb_w],
                            axis=0)
    return _run(token_ids, action_actors, action_streets, action_legal_masks,
                table, mlp_w, mlp_b, ln_gamma, ln_beta)


# speedup vs baseline: 3.7105x; 1.0475x over previous
import functools

import jax
import jax.numpy as jnp
from jax import lax
from jax.experimental import pallas as pl

_NUM_BET_BINS = 32
_D_MODEL = 128
_NUM_STREETS = 4
_ACTION_OFFSET = 10


def _fused_kernel(tok_ref, act_ref, st_ref, masks_ref, table_ref, mlp_w_ref,
                  mlp_b_ref, gamma_ref, beta_ref, out_ref):
    tok = tok_ref[...]
    act = act_ref[...]
    st = st_ref[...]

    r = tok.shape[0]
    valid = ((tok >= _ACTION_OFFSET)
             & (tok < _ACTION_OFFSET + _NUM_BET_BINS)).astype(jnp.float32)
    aid = jnp.clip(tok - _ACTION_OFFSET, 0, _NUM_BET_BINS - 1)
    act = jnp.clip(act, 0, 1)
    st = jnp.clip(st, 0, _NUM_STREETS - 1)

    i38 = lax.broadcasted_iota(jnp.int32, (r, 38), 1)
    oh = jnp.where(i38 < 2, (act == i38).astype(jnp.float32), 0.0)
    oh = jnp.where((i38 >= 2) & (i38 < 6),
                   (st == i38 - 2).astype(jnp.float32), oh)
    oh = jnp.where(i38 >= 6, (aid == i38 - 6).astype(jnp.float32), oh)

    emb = jnp.dot(oh, table_ref[...], preferred_element_type=jnp.float32)

    h = jnp.dot(masks_ref[...], mlp_w_ref[...],
                preferred_element_type=jnp.float32) + mlp_b_ref[...]
    m = jnp.mean(h, axis=1, keepdims=True)
    c = h - m
    v = jnp.mean(c * c, axis=1, keepdims=True)
    h = c * lax.rsqrt(v + 1e-5) * gamma_ref[...] + beta_ref[...]
    h = jnp.maximum(h, 0.0)

    out_ref[...] = valid * (emb + h)


@functools.partial(jax.jit, static_argnames=("block_rows",))
def _run(token_ids, action_actors, action_streets, action_legal_masks,
         table, mlp_w, mlp_b, ln_gamma, ln_beta, block_rows=2048):
    b, l = token_ids.shape
    n = b * l
    tok = token_ids.reshape(n, 1).astype(jnp.int32)
    act = action_actors.reshape(n, 1).astype(jnp.int32)
    st = action_streets.reshape(n, 1).astype(jnp.int32)
    masks = action_legal_masks.reshape(n, _NUM_BET_BINS)

    grid = n // block_rows
    row_spec = pl.BlockSpec((block_rows, 1), lambda i: (i, 0))
    full = lambda shape: pl.BlockSpec(shape, lambda i: (0, 0))

    out = pl.pallas_call(
        _fused_kernel,
        grid=(grid,),
        in_specs=[
            row_spec, row_spec, row_spec,
            pl.BlockSpec((block_rows, _NUM_BET_BINS), lambda i: (i, 0)),
            full(table.shape),
            full(mlp_w.shape),
            full((1, _D_MODEL)),
            full((1, _D_MODEL)),
            full((1, _D_MODEL)),
        ],
        out_specs=pl.BlockSpec((block_rows, _D_MODEL), lambda i: (i, 0)),
        out_shape=jax.ShapeDtypeStruct((n, _D_MODEL), jnp.float32),
    )(tok, act, st, masks, table, mlp_w,
      mlp_b.reshape(1, _D_MODEL), ln_gamma.reshape(1, _D_MODEL),
      ln_beta.reshape(1, _D_MODEL))
    return out.reshape(b, l, _D_MODEL)


def kernel(token_ids, action_actors, action_streets, action_legal_masks,
           actor_emb_w, street_emb_w, action_type_emb_w, mlp_w, mlp_b,
           ln_gamma, ln_beta):
    table = jnp.concatenate([actor_emb_w, street_emb_w, action_type_emb_w],
                            axis=0)
    return _run(token_ids, action_actors, action_streets, action_legal_masks,
                table, mlp_w, mlp_b, ln_gamma, ln_beta, block_rows=8192)
